# baseline (speedup 1.0000x reference)
import functools

import jax
import jax.numpy as jnp
from jax import lax
from jax.experimental import pallas as pl
from jax.experimental.pallas import tpu as pltpu
from jax.experimental.pallas import tpu_sc as plsc

_N = 10000
_D = 128
_H = 64
_W = 128
_E = 320000

_NC = 2
_NS = 16
_NW = _NC * _NS

_BPW = _E // _NW
_CB = 32
_NFULL = _BPW // _CB
_TAIL = _BPW - _NFULL * _CB


def _mm_body(x_ref, wt_ref, b_ref, o_ref):
    h = (
        jnp.dot(x_ref[...], wt_ref[...], preferred_element_type=jnp.float32)
        + b_ref[...]
    )
    o_ref[:, 0:_H] = h
    o_ref[:, _H:_W] = h


def _project_dup(features, W, b):
    wt = W.T
    return pl.pallas_call(
        _mm_body,
        grid=(10,),
        in_specs=[
            pl.BlockSpec((_N // 10, _D), lambda i: (i, 0)),
            pl.BlockSpec((_D, _H), lambda i: (0, 0)),
            pl.BlockSpec((1, _H), lambda i: (0, 0)),
        ],
        out_specs=pl.BlockSpec((_N // 10, _W), lambda i: (i, 0)),
        out_shape=jax.ShapeDtypeStruct((_N, _W), jnp.float32),
    )(features, wt, b.reshape(1, _H))


_mesh = plsc.VectorSubcoreMesh(core_axis_name="c", subcore_axis_name="s")


@functools.partial(
    pl.kernel,
    mesh=_mesh,
    out_type=[
        jax.ShapeDtypeStruct((_E // 2, _W), jnp.float32),
        jax.ShapeDtypeStruct((_E // 2, _W), jnp.float32),
    ],
    scratch_types=[
        pltpu.VMEM((_BPW,), jnp.int32),
        pltpu.VMEM((_BPW,), jnp.int32),
        pltpu.VMEM((_CB, _W), jnp.float32),
        pltpu.VMEM((_CB, _W), jnp.float32),
        pltpu.VMEM((_CB, _W), jnp.float32),
        pltpu.VMEM((_CB, _W), jnp.float32),
        pltpu.VMEM((_CB // 2, _W), jnp.float32),
        pltpu.VMEM((_CB // 2, _W), jnp.float32),
        pltpu.VMEM_SHARED((_N, _W), jnp.float32),
        pltpu.SemaphoreType.DMA,
        pltpu.SemaphoreType.DMA,
        pltpu.SemaphoreType.DMA,
        pltpu.SemaphoreType.DMA,
    ],
)
def _edge_sc(h_hbm, s0_hbm, d0_hbm, s1_hbm, d1_hbm, f0_hbm, f1_hbm,
             idx_s, idx_d, rs0, rs1, rd0, rd1, ro0, ro1, h_sp,
             sg0, sg1, so0, so1):
    wid = lax.axis_index("s") * _NC + lax.axis_index("c")
    ebase = wid * _BPW

    sid = lax.axis_index("s")
    rows_per_tile = (_N // _NS) // 8 * 8
    tail_rows = _N - rows_per_tile * _NS
    stage_off = pl.multiple_of(sid * rows_per_tile, 8)
    pltpu.sync_copy(
        h_hbm.at[pl.ds(stage_off, rows_per_tile)],
        h_sp.at[pl.ds(stage_off, rows_per_tile)],
    )

    @pl.when(sid == 0)
    def _copy_tail():
        pltpu.sync_copy(
            h_hbm.at[pl.ds(rows_per_tile * _NS, tail_rows)],
            h_sp.at[pl.ds(rows_per_tile * _NS, tail_rows)],
        )

    plsc.subcore_barrier()

    rs = (rs0, rs1)
    rd = (rd0, rd1)
    ro = (ro0, ro1)
    sg = (sg0, sg1)
    so = (so0, so1)

    def do_rel(s_hbm, d_hbm, f_hbm):
        ib = pl.multiple_of(ebase, 8)
        pltpu.sync_copy(s_hbm.at[pl.ds(ib, _BPW)], idx_s)
        pltpu.sync_copy(d_hbm.at[pl.ds(ib, _BPW)], idx_d)

        def issue_gather(k, buf):
            pltpu.async_copy(h_sp.at[idx_s.at[pl.ds(k * _CB, _CB)]], rs[buf], sg[buf])
            pltpu.async_copy(h_sp.at[idx_d.at[pl.ds(k * _CB, _CB)]], rd[buf], sg[buf])

        def drain_gather(buf):
            pltpu.make_async_copy(h_hbm.at[pl.ds(0, _CB)], rs[buf], sg[buf]).wait()
            pltpu.make_async_copy(h_hbm.at[pl.ds(0, _CB)], rd[buf], sg[buf]).wait()

        def compute(buf, nedge):
            unroll = 2

            def pair_body(it, _):
                for u in range(unroll):
                    r = it * unroll + u
                    for half in range(2):
                        e = 2 * r + half
                        for j in range(_H // 16):
                            a = rs[buf][e, pl.ds(j * 16, 16)]
                            bb = rd[buf][e, pl.ds(j * 16, 16)]
                            ro[buf][r, pl.ds(half * _H + j * 16, 16)] = jnp.abs(a - bb)
                return 0

            lax.fori_loop(0, nedge // 2 // unroll, pair_body, 0)

        rbase = ebase // 2
        _RB = _CB // 2

        def issue_out(k, buf):
            off = pl.multiple_of(rbase + k * _RB, 8)
            pltpu.async_copy(ro[buf], f_hbm.at[pl.ds(off, _RB)], so[buf])

        def drain_out(buf):
            pltpu.make_async_copy(f_hbm.at[pl.ds(0, _RB)], ro[buf], so[buf]).wait()

        def half(i, k, buf):
            drain_gather(buf)

            @pl.when(i > 0)
            def _drain_prev_out():
                drain_out(buf)

            compute(buf, _CB)
            issue_out(k, buf)

            @pl.when(k + 2 < _NFULL)
            def _prefetch():
                issue_gather(k + 2, buf)

        issue_gather(0, 0)
        issue_gather(1, 1)

        def pair_body(i, _):
            half(i, 2 * i, 0)
            half(i, 2 * i + 1, 1)
            return 0

        lax.fori_loop(0, _NFULL // 2, pair_body, 0)

        toff = _NFULL * _CB
        pltpu.async_copy(
            h_sp.at[idx_s.at[pl.ds(toff, _TAIL)]], rs0.at[pl.ds(0, _TAIL)], sg0)
        pltpu.async_copy(
            h_sp.at[idx_d.at[pl.ds(toff, _TAIL)]], rd0.at[pl.ds(0, _TAIL)], sg0)
        pltpu.make_async_copy(
            h_hbm.at[pl.ds(0, _TAIL)], rs0.at[pl.ds(0, _TAIL)], sg0).wait()
        pltpu.make_async_copy(
            h_hbm.at[pl.ds(0, _TAIL)], rd0.at[pl.ds(0, _TAIL)], sg0).wait()
        drain_out(0)

        def tail_body(r, _):
            for half in range(2):
                e = 2 * r + half
                for j in range(_H // 16):
                    a = rs0[e, pl.ds(j * 16, 16)]
                    bb = rd0[e, pl.ds(j * 16, 16)]
                    ro0[r, pl.ds(half * _H + j * 16, 16)] = jnp.abs(a - bb)
            return 0

        _TR = _TAIL // 2
        lax.fori_loop(0, _TR, tail_body, 0)
        tout = pl.multiple_of(rbase + toff // 2, 8)
        pltpu.async_copy(
            ro0.at[pl.ds(0, _TR)], f_hbm.at[pl.ds(tout, _TR)], so0)

        pltpu.make_async_copy(
            f_hbm.at[pl.ds(0, _TR)], ro0.at[pl.ds(0, _TR)], so0).wait()
        drain_out(1)

    do_rel(s0_hbm, d0_hbm, f0_hbm)
    do_rel(s1_hbm, d1_hbm, f1_hbm)


def kernel(features, W, b, edge_index0, edge_index1, labels0, labels1):
    hdup = _project_dup(features, W, b)
    f0p, f1p = _edge_sc(
        hdup,
        edge_index0[0], edge_index0[1],
        edge_index1[0], edge_index1[1],
    )
    f0 = f0p.reshape(_E, _H)
    f1 = f1p.reshape(_E, _H)
    return (edge_index0, edge_index1, f0, f1, labels0, labels1)


# speedup vs baseline: 1.9795x; 1.9795x over previous
import functools

import jax
import jax.numpy as jnp
from jax import lax
from jax.experimental import pallas as pl
from jax.experimental.pallas import tpu as pltpu
from jax.experimental.pallas import tpu_sc as plsc

_N = 10000
_D = 128
_H = 64
_W = 128
_E = 320000

_NC = 2
_NS = 16
_NW = _NC * _NS

_BPW = _E // _NW
_CB = 32
_NFULL = _BPW // _CB
_TAIL = _BPW - _NFULL * _CB


def _mm_body(x_ref, wt_ref, b_ref, o_ref):
    h = (
        jnp.dot(x_ref[...], wt_ref[...], preferred_element_type=jnp.float32)
        + b_ref[...]
    )
    o_ref[:, 0:_H] = h
    o_ref[:, _H:_W] = h


def _project_dup(features, W, b):
    wt = W.T
    return pl.pallas_call(
        _mm_body,
        grid=(10,),
        in_specs=[
            pl.BlockSpec((_N // 10, _D), lambda i: (i, 0)),
            pl.BlockSpec((_D, _H), lambda i: (0, 0)),
            pl.BlockSpec((1, _H), lambda i: (0, 0)),
        ],
        out_specs=pl.BlockSpec((_N // 10, _W), lambda i: (i, 0)),
        out_shape=jax.ShapeDtypeStruct((_N, _W), jnp.float32),
    )(features, wt, b.reshape(1, _H))


_mesh = plsc.VectorSubcoreMesh(core_axis_name="c", subcore_axis_name="s")


@functools.partial(
    pl.kernel,
    mesh=_mesh,
    out_type=jax.ShapeDtypeStruct((_E, _H), jnp.float32),
    scratch_types=[
        pltpu.VMEM((_BPW,), jnp.int32),
        pltpu.VMEM((_BPW,), jnp.int32),
        pltpu.VMEM((_CB, _W), jnp.float32),
        pltpu.VMEM((_CB, _W), jnp.float32),
        pltpu.VMEM((_CB, _W), jnp.float32),
        pltpu.VMEM((_CB, _W), jnp.float32),
        pltpu.VMEM((_CB, _H), jnp.float32),
        pltpu.VMEM((_CB, _H), jnp.float32),
        pltpu.VMEM_SHARED((_N, _W), jnp.float32),
        pltpu.SemaphoreType.DMA,
        pltpu.SemaphoreType.DMA,
        pltpu.SemaphoreType.DMA,
        pltpu.SemaphoreType.DMA,
    ],
)
def _edge_sc(h_hbm, s_hbm, d_hbm, f_hbm,
             idx_s, idx_d, rs0, rs1, rd0, rd1, ro0, ro1, h_sp,
             sg0, sg1, so0, so1):
    wid = lax.axis_index("s") * _NC + lax.axis_index("c")
    ebase = wid * _BPW

    sid = lax.axis_index("s")
    rows_per_tile = (_N // _NS) // 8 * 8
    tail_rows = _N - rows_per_tile * _NS
    pltpu.sync_copy(
        h_hbm.at[pl.ds(sid * rows_per_tile, rows_per_tile)],
        h_sp.at[pl.ds(sid * rows_per_tile, rows_per_tile)],
    )

    @pl.when(sid == 0)
    def _copy_tail():
        pltpu.sync_copy(
            h_hbm.at[pl.ds(rows_per_tile * _NS, tail_rows)],
            h_sp.at[pl.ds(rows_per_tile * _NS, tail_rows)],
        )

    plsc.subcore_barrier()

    rs = (rs0, rs1)
    rd = (rd0, rd1)
    ro = (ro0, ro1)
    sg = (sg0, sg1)
    so = (so0, so1)

    def do_rel(s_hbm, d_hbm, f_hbm):
        pltpu.sync_copy(s_hbm.at[pl.ds(ebase, _BPW)], idx_s)
        pltpu.sync_copy(d_hbm.at[pl.ds(ebase, _BPW)], idx_d)

        def issue_gather(k, buf):
            pltpu.async_copy(h_sp.at[idx_s.at[pl.ds(k * _CB, _CB)]], rs[buf], sg[buf])
            pltpu.async_copy(h_sp.at[idx_d.at[pl.ds(k * _CB, _CB)]], rd[buf], sg[buf])

        def drain_gather(buf):
            pltpu.make_async_copy(h_hbm.at[pl.ds(0, _CB)], rs[buf], sg[buf]).wait()
            pltpu.make_async_copy(h_hbm.at[pl.ds(0, _CB)], rd[buf], sg[buf]).wait()

        def compute(buf, nedge):
            unroll = 4

            def edge_body(it, _):
                for u in range(unroll):
                    e = it * unroll + u
                    for j in range(_H // 16):
                        a = rs[buf][e, pl.ds(j * 16, 16)]
                        bb = rd[buf][e, pl.ds(j * 16, 16)]
                        ro[buf][e, pl.ds(j * 16, 16)] = jnp.abs(a - bb)
                return 0

            lax.fori_loop(0, nedge // unroll, edge_body, 0)

        def issue_out(k, buf):
            pltpu.async_copy(ro[buf], f_hbm.at[pl.ds(ebase + k * _CB, _CB)], so[buf])

        def drain_out(buf):
            pltpu.make_async_copy(f_hbm.at[pl.ds(0, _CB)], ro[buf], so[buf]).wait()

        def half(i, k, buf):
            drain_gather(buf)

            @pl.when(i > 0)
            def _drain_prev_out():
                drain_out(buf)

            compute(buf, _CB)
            issue_out(k, buf)

            @pl.when(k + 2 < _NFULL)
            def _prefetch():
                issue_gather(k + 2, buf)

        issue_gather(0, 0)
        issue_gather(1, 1)

        def pair_body(i, _):
            half(i, 2 * i, 0)
            half(i, 2 * i + 1, 1)
            return 0

        lax.fori_loop(0, _NFULL // 2, pair_body, 0)

        toff = _NFULL * _CB
        pltpu.async_copy(
            h_sp.at[idx_s.at[pl.ds(toff, _TAIL)]], rs0.at[pl.ds(0, _TAIL)], sg0)
        pltpu.async_copy(
            h_sp.at[idx_d.at[pl.ds(toff, _TAIL)]], rd0.at[pl.ds(0, _TAIL)], sg0)
        pltpu.make_async_copy(
            h_hbm.at[pl.ds(0, _TAIL)], rs0.at[pl.ds(0, _TAIL)], sg0).wait()
        pltpu.make_async_copy(
            h_hbm.at[pl.ds(0, _TAIL)], rd0.at[pl.ds(0, _TAIL)], sg0).wait()
        drain_out(0)

        def tail_body(e, _):
            for j in range(_H // 16):
                a = rs0[e, pl.ds(j * 16, 16)]
                bb = rd0[e, pl.ds(j * 16, 16)]
                ro0[e, pl.ds(j * 16, 16)] = jnp.abs(a - bb)
            return 0

        lax.fori_loop(0, _TAIL, tail_body, 0)
        pltpu.async_copy(
            ro0.at[pl.ds(0, _TAIL)], f_hbm.at[pl.ds(ebase + toff, _TAIL)], so0)

        pltpu.make_async_copy(
            f_hbm.at[pl.ds(0, _TAIL)], ro0.at[pl.ds(0, _TAIL)], so0).wait()
        drain_out(1)

    do_rel(s_hbm, d_hbm, f_hbm)


def kernel(features, W, b, edge_index0, edge_index1, labels0, labels1):
    hdup = _project_dup(features, W, b)
    f0 = _edge_sc(hdup, edge_index0[0], edge_index0[1])
    f1 = _edge_sc(hdup, edge_index1[0], edge_index1[1])
    return (edge_index0, edge_index1, f0, f1, labels0, labels1)
